# speedup vs baseline: 2.9625x; 2.9625x over previous
import functools

import jax
import jax.numpy as jnp
from jax import lax
from jax.experimental import pallas as pl
from jax.experimental.pallas import tpu as pltpu
from jax.experimental.pallas import tpu_sc as plsc

NC = 2
NS = 16
NW = NC * NS

C = 128


def _make_sc_gather(B, D, n_chunks):
    mesh = plsc.VectorSubcoreMesh(core_axis_name="c", subcore_axis_name="s")

    @functools.partial(
        pl.kernel,
        mesh=mesh,
        out_type=jax.ShapeDtypeStruct((B, D), jnp.float32),
        scratch_types=[
            pltpu.VMEM((n_chunks, C), jnp.int32),
            pltpu.VMEM((C, D), jnp.float32),
            pltpu.SemaphoreType.DMA,
        ],
    )
    def k(table_hbm, idx_hbm, out_hbm, idx_v, rows_v, sem):
        wid = lax.axis_index("s") * NC + lax.axis_index("c")
        b_per_w = n_chunks * C
        base = wid * b_per_w
        pltpu.sync_copy(idx_hbm.at[wid], idx_v)

        def body(i, _):
            pltpu.async_copy(table_hbm.at[idx_v.at[i]], rows_v, sem).wait()
            pltpu.sync_copy(rows_v, out_hbm.at[pl.ds(base + i * C, C)])
            return 0

        lax.fori_loop(0, n_chunks, body, 0)

    return k


def kernel(seq, table):
    S0, S1 = seq.shape
    B = S0 * S1
    D = table.shape[1]
    assert B % (NW * C) == 0
    n_chunks = B // (NW * C)
    idx = seq.reshape(NW, n_chunks, C).astype(jnp.int32)
    out = _make_sc_gather(B, D, n_chunks)(table, idx)
    return out.reshape(S0, S1, D)


# speedup vs baseline: 3.2974x; 1.1131x over previous
import functools

import jax
import jax.numpy as jnp
from jax import lax
from jax.experimental import pallas as pl
from jax.experimental.pallas import tpu as pltpu
from jax.experimental.pallas import tpu_sc as plsc

NC = 2
NS = 16
NW = NC * NS

C = 128
NBUF = 5


def _make_sc_gather(B, D, n_chunks):
    assert n_chunks % NBUF == 0
    n_groups = n_chunks // NBUF
    mesh = plsc.VectorSubcoreMesh(core_axis_name="c", subcore_axis_name="s")

    scratch = [pltpu.VMEM((n_chunks, C), jnp.int32)]
    scratch += [pltpu.VMEM((C, D), jnp.float32) for _ in range(NBUF)]
    scratch += [pltpu.SemaphoreType.DMA for _ in range(2 * NBUF)]

    @functools.partial(
        pl.kernel,
        mesh=mesh,
        out_type=jax.ShapeDtypeStruct((B, D), jnp.float32),
        scratch_types=scratch,
    )
    def k(table_hbm, idx_hbm, out_hbm, idx_v, *rest):
        rows = rest[:NBUF]
        gsem = rest[NBUF:2 * NBUF]
        wsem = rest[2 * NBUF:]
        wid = lax.axis_index("s") * NC + lax.axis_index("c")
        base = wid * n_chunks * C
        pltpu.sync_copy(idx_hbm.at[wid], idx_v)

        def start_gather(i, b):
            pltpu.async_copy(table_hbm.at[idx_v.at[i]], rows[b], gsem[b])

        def wait_gather(b):
            pltpu.make_async_copy(
                table_hbm.at[idx_v.at[0]], rows[b], gsem[b]).wait()

        def start_write(i, b):
            pltpu.async_copy(
                rows[b], out_hbm.at[pl.ds(base + i * C, C)], wsem[b])

        def wait_write(b):
            pltpu.make_async_copy(
                rows[b], out_hbm.at[pl.ds(base, C)], wsem[b]).wait()

        for b in range(NBUF):
            start_gather(b, b)
        for b in range(NBUF):
            wait_gather(b)
            start_write(b, b)

        def body(j, _):
            i0 = (j + 1) * NBUF
            for b in range(NBUF):
                wait_write(b)
                start_gather(i0 + b, b)
            for b in range(NBUF):
                wait_gather(b)
                start_write(i0 + b, b)
            return 0

        lax.fori_loop(0, n_groups - 1, body, 0)
        for b in range(NBUF):
            wait_write(b)

    return k


def kernel(seq, table):
    S0, S1 = seq.shape
    B = S0 * S1
    D = table.shape[1]
    assert B % (NW * C) == 0
    n_chunks = B // (NW * C)
    idx = seq.reshape(NW, n_chunks, C).astype(jnp.int32)
    out = _make_sc_gather(B, D, n_chunks)(table, idx)
    return out.reshape(S0, S1, D)


# speedup vs baseline: 5.8929x; 1.7871x over previous
import functools

import jax
import jax.numpy as jnp
from jax import lax
from jax.experimental import pallas as pl
from jax.experimental.pallas import tpu as pltpu
from jax.experimental.pallas import tpu_sc as plsc

NC = 2
NS = 16
NW = NC * NS

PAD = 56
NBUF = 8


def _make_sc_gather(S0, T, D):
    seq_per_w = S0 // NW
    assert seq_per_w % NBUF == 0
    n_groups = seq_per_w // NBUF
    mesh = plsc.VectorSubcoreMesh(core_axis_name="c", subcore_axis_name="s")

    scratch = [pltpu.VMEM((seq_per_w * PAD,), jnp.int32)]
    scratch += [pltpu.VMEM((T, D), jnp.float32) for _ in range(NBUF)]
    scratch += [pltpu.SemaphoreType.DMA for _ in range(2 * NBUF)]

    @functools.partial(
        pl.kernel,
        mesh=mesh,
        out_type=jax.ShapeDtypeStruct((S0, T, D), jnp.float32),
        scratch_types=scratch,
        compiler_params=pltpu.CompilerParams(use_tc_tiling_on_sc=True),
    )
    def k(table_hbm, idx_hbm, out_hbm, idx_v, *rest):
        rows = rest[:NBUF]
        gsem = rest[NBUF:2 * NBUF]
        wsem = rest[2 * NBUF:]
        wid = lax.axis_index("s") * NC + lax.axis_index("c")
        s0 = wid * seq_per_w
        pltpu.sync_copy(
            idx_hbm.at[pl.ds(s0 * PAD, seq_per_w * PAD)], idx_v)

        def idx_slice(j):
            return idx_v.at[pl.ds(pl.multiple_of(j * PAD, 8), T)]

        def start_gather(j, b):
            pltpu.async_copy(table_hbm.at[idx_slice(j)], rows[b], gsem[b])

        def wait_gather(b):
            pltpu.make_async_copy(
                table_hbm.at[idx_slice(0)], rows[b], gsem[b]).wait()

        def start_write(j, b):
            pltpu.async_copy(rows[b], out_hbm.at[s0 + j], wsem[b])

        def wait_write(b):
            pltpu.make_async_copy(rows[b], out_hbm.at[s0], wsem[b]).wait()

        for b in range(NBUF):
            start_gather(b, b)
        for b in range(NBUF):
            wait_gather(b)
            start_write(b, b)

        def body(g, _):
            j0 = (g + 1) * NBUF
            for b in range(NBUF):
                wait_write(b)
                start_gather(j0 + b, b)
            for b in range(NBUF):
                wait_gather(b)
                start_write(j0 + b, b)
            return 0

        lax.fori_loop(0, n_groups - 1, body, 0)
        for b in range(NBUF):
            wait_write(b)

    return k


def kernel(seq, table):
    S0, T = seq.shape
    D = table.shape[1]
    assert S0 % NW == 0 and T <= PAD
    idx = jnp.pad(seq.astype(jnp.int32), ((0, 0), (0, PAD - T))).reshape(-1)
    return _make_sc_gather(S0, T, D)(table, idx)


# speedup vs baseline: 10.1204x; 1.7174x over previous
import functools

import jax
import jax.numpy as jnp
from jax import lax
from jax.experimental import pallas as pl
from jax.experimental.pallas import tpu as pltpu
from jax.experimental.pallas import tpu_sc as plsc

NC = 2
NS = 16
NW = NC * NS

C = 128
NBUF = 5
PADC = 56


def _make_sc_gather(B, D, n_chunks):
    assert n_chunks % NBUF == 0
    n_groups = n_chunks // NBUF
    mesh = plsc.VectorSubcoreMesh(core_axis_name="c", subcore_axis_name="s")

    scratch = [pltpu.VMEM((PADC, C), jnp.int32)]
    scratch += [pltpu.VMEM((C, D), jnp.float32) for _ in range(NBUF)]
    scratch += [pltpu.SemaphoreType.DMA for _ in range(2 * NBUF)]

    @functools.partial(
        pl.kernel,
        mesh=mesh,
        out_type=jax.ShapeDtypeStruct((B, D), jnp.float32),
        scratch_types=scratch,
    )
    def k(table_hbm, idx_hbm, out_hbm, idx_v, *rest):
        rows = rest[:NBUF]
        gsem = rest[NBUF:2 * NBUF]
        wsem = rest[2 * NBUF:]
        wid = lax.axis_index("s") * NC + lax.axis_index("c")
        base = wid * n_chunks * C
        pltpu.sync_copy(
            idx_hbm.at[pl.ds(pl.multiple_of(wid * PADC, 8), PADC)], idx_v)

        def start_gather(i, b):
            pltpu.async_copy(table_hbm.at[idx_v.at[i]], rows[b], gsem[b])

        def wait_gather(b):
            pltpu.make_async_copy(
                table_hbm.at[idx_v.at[0]], rows[b], gsem[b]).wait()

        def start_write(i, b):
            pltpu.async_copy(
                rows[b], out_hbm.at[pl.ds(base + i * C, C)], wsem[b])

        def wait_write(b):
            pltpu.make_async_copy(
                rows[b], out_hbm.at[pl.ds(base, C)], wsem[b]).wait()

        for b in range(NBUF):
            start_gather(b, b)
        for b in range(NBUF):
            wait_gather(b)
            start_write(b, b)

        def body(j, _):
            i0 = (j + 1) * NBUF
            for b in range(NBUF):
                wait_write(b)
                start_gather(i0 + b, b)
            for b in range(NBUF):
                wait_gather(b)
                start_write(i0 + b, b)
            return 0

        lax.fori_loop(0, n_groups - 1, body, 0)
        for b in range(NBUF):
            wait_write(b)

    return k


def kernel(seq, table):
    S0, T = seq.shape
    B = S0 * T
    D = table.shape[1]
    assert B % (NW * C) == 0
    n_chunks = B // (NW * C)
    idx = jnp.pad(
        seq.T.astype(jnp.int32).reshape(NW, n_chunks, C),
        ((0, 0), (0, PADC - n_chunks), (0, 0))).reshape(NW * PADC, C)
    out = _make_sc_gather(B, D, n_chunks)(table, idx)
    return out.reshape(T, S0, D).transpose(1, 0, 2)


# speedup vs baseline: 10.3147x; 1.0192x over previous
import functools

import jax
import jax.numpy as jnp
from jax import lax
from jax.experimental import pallas as pl
from jax.experimental.pallas import tpu as pltpu
from jax.experimental.pallas import tpu_sc as plsc

NC = 2
NS = 16
NW = NC * NS

C = 64
NBUF = 10


def _make_sc_gather(B, D, n_chunks):
    assert n_chunks % NBUF == 0
    n_groups = n_chunks // NBUF
    b_per_w = n_chunks * C
    mesh = plsc.VectorSubcoreMesh(core_axis_name="c", subcore_axis_name="s")

    scratch = [pltpu.VMEM((b_per_w,), jnp.int32)]
    scratch += [pltpu.VMEM((C, D), jnp.float32) for _ in range(NBUF)]
    scratch += [pltpu.SemaphoreType.DMA for _ in range(2 * NBUF)]

    @functools.partial(
        pl.kernel,
        mesh=mesh,
        out_type=jax.ShapeDtypeStruct((B, D), jnp.float32),
        scratch_types=scratch,
    )
    def k(table_hbm, idx_hbm, out_hbm, idx_v, *rest):
        rows = rest[:NBUF]
        gsem = rest[NBUF:2 * NBUF]
        wsem = rest[2 * NBUF:]
        wid = lax.axis_index("s") * NC + lax.axis_index("c")
        base = wid * b_per_w
        pltpu.sync_copy(
            idx_hbm.at[pl.ds(pl.multiple_of(base, 8), b_per_w)], idx_v)

        def idx_slice(i):
            return idx_v.at[pl.ds(pl.multiple_of(i * C, 8), C)]

        def start_gather(i, b):
            pltpu.async_copy(table_hbm.at[idx_slice(i)], rows[b], gsem[b])

        def wait_gather(b):
            pltpu.make_async_copy(
                table_hbm.at[idx_slice(0)], rows[b], gsem[b]).wait()

        def start_write(i, b):
            pltpu.async_copy(
                rows[b], out_hbm.at[pl.ds(base + i * C, C)], wsem[b])

        def wait_write(b):
            pltpu.make_async_copy(
                rows[b], out_hbm.at[pl.ds(base, C)], wsem[b]).wait()

        for b in range(NBUF):
            start_gather(b, b)
        for b in range(NBUF):
            wait_gather(b)
            start_write(b, b)

        def body(j, _):
            i0 = (j + 1) * NBUF
            for b in range(NBUF):
                wait_write(b)
                start_gather(i0 + b, b)
            for b in range(NBUF):
                wait_gather(b)
                start_write(i0 + b, b)
            return 0

        lax.fori_loop(0, n_groups - 1, body, 0)
        for b in range(NBUF):
            wait_write(b)

    return k


def kernel(seq, table):
    S0, T = seq.shape
    B = S0 * T
    D = table.shape[1]
    assert B % (NW * C) == 0
    n_chunks = B // (NW * C)
    idx = seq.T.astype(jnp.int32).reshape(B)
    out = _make_sc_gather(B, D, n_chunks)(table, idx)
    return out.reshape(T, S0, D).transpose(1, 0, 2)
